# baseline (speedup 1.0000x reference)
import functools

import jax
import jax.numpy as jnp
from jax import lax
from jax.experimental import pallas as pl
from jax.experimental.pallas import tpu as pltpu
from jax.experimental.pallas import tpu_sc as plsc

N = 10000
E = 320000
D_IN = 128
H = 128
HH = H // 2
N_OUT = 2

NP = 10240
NC = 2
NS = 16
NW = NC * NS

CHUNK = 128
NCHUNK = 160
EPT = NCHUNK * CHUNK
EP = NS * EPT
NBUF = 4
RPT = NP // NS

DCHUNK = 80
DNCHUNK = 128
DEPT = DCHUNK * DNCHUNK

_sc_mesh = plsc.VectorSubcoreMesh(core_axis_name="c", subcore_axis_name="s")


def _deg_body(dst_hbm, zeros_hbm, out_hbm, didx_v, ones_v, acc_sh):
    c = lax.axis_index("c")
    s = lax.axis_index("s")
    wid = s * NC + c
    pltpu.sync_copy(zeros_hbm.at[pl.ds(s * RPT, RPT)], acc_sh.at[pl.ds(s * RPT, RPT)])
    pltpu.sync_copy(dst_hbm.at[wid], didx_v)
    for g in range(DCHUNK // 16):
        ones_v[pl.ds(g * 16, 16)] = jnp.ones((16,), jnp.float32)
    plsc.subcore_barrier()

    def step(i, carry):
        pltpu.sync_copy(ones_v, acc_sh.at[didx_v.at[i]], add=True)
        return carry

    lax.fori_loop(0, DNCHUNK, step, 0)
    plsc.subcore_barrier()
    pltpu.sync_copy(acc_sh.at[pl.ds(s * RPT, RPT)], out_hbm.at[c, pl.ds(s * RPT, RPT)])


_deg_call = pl.kernel(
    _deg_body,
    out_type=jax.ShapeDtypeStruct((NC, NP), jnp.float32),
    mesh=_sc_mesh,
    scratch_types=[
        pltpu.VMEM((DNCHUNK, DCHUNK), jnp.int32),
        pltpu.VMEM((DCHUNK,), jnp.float32),
        pltpu.VMEM_SHARED((NP,), jnp.float32),
    ],
)


def _agg_body(src_hbm, dst_hbm, tab0_hbm, tab1_hbm, zeros_hbm, out_hbm,
              sidx_v, didx_v, rows_v, acc_sh, gsems, ssems):
    c = lax.axis_index("c")
    s = lax.axis_index("s")
    pltpu.sync_copy(zeros_hbm.at[pl.ds(s * RPT, RPT)], acc_sh.at[pl.ds(s * RPT, RPT)])
    pltpu.sync_copy(src_hbm.at[s], sidx_v)
    pltpu.sync_copy(dst_hbm.at[s], didx_v)
    plsc.subcore_barrier()

    def gather(i, b):
        @pl.when(c == 0)
        def _():
            pltpu.async_copy(
                tab0_hbm.at[sidx_v.at[pl.ds(i * CHUNK, CHUNK)]],
                rows_v.at[b], gsems.at[b])

        @pl.when(c == 1)
        def _():
            pltpu.async_copy(
                tab1_hbm.at[sidx_v.at[pl.ds(i * CHUNK, CHUNK)]],
                rows_v.at[b], gsems.at[b])

    def wait_g(i, b):
        pltpu.make_async_copy(
            tab0_hbm.at[sidx_v.at[pl.ds(i * CHUNK, CHUNK)]],
            rows_v.at[b], gsems.at[b]).wait()

    def scat(i, b):
        pltpu.async_copy(
            rows_v.at[b], acc_sh.at[didx_v.at[i]], ssems.at[b], add=True)

    def wait_s(i, b):
        pltpu.make_async_copy(
            rows_v.at[b], acc_sh.at[didx_v.at[i]], ssems.at[b]).wait()

    for b in range(NBUF):
        gather(b, b)

    def turn(j, carry):
        i0 = j * NBUF
        for b in range(NBUF):
            wait_g(i0 + b, b)
            scat(i0 + b, b)
        for b in range(NBUF):
            wait_s(i0 + b, b)
            gather(i0 + NBUF + b, b)
        return carry

    lax.fori_loop(0, NCHUNK // NBUF - 1, turn, 0)
    i0 = NCHUNK - NBUF
    for b in range(NBUF):
        wait_g(i0 + b, b)
        scat(i0 + b, b)
    for b in range(NBUF):
        wait_s(i0 + b, b)
    plsc.subcore_barrier()
    pltpu.sync_copy(acc_sh.at[pl.ds(s * RPT, RPT)], out_hbm.at[c, pl.ds(s * RPT, RPT)])


_agg_call = pl.kernel(
    _agg_body,
    out_type=jax.ShapeDtypeStruct((NC, NP, HH), jnp.float32),
    mesh=_sc_mesh,
    scratch_types=[
        pltpu.VMEM((EPT,), jnp.int32),
        pltpu.VMEM((NCHUNK, CHUNK), jnp.int32),
        pltpu.VMEM((NBUF, CHUNK, HH), jnp.float32),
        pltpu.VMEM_SHARED((NP, HH), jnp.float32),
        pltpu.SemaphoreType.DMA((NBUF,)),
        pltpu.SemaphoreType.DMA((NBUF,)),
    ],
    compiler_params=pltpu.CompilerParams(use_tc_tiling_on_sc=False),
)

ROWS_B = 1000


def _k1_body(x_ref, w0_ref, b0_ref, wc0_ref, d0_ref, d1_ref,
             ta_ref, tb_ref, tself_ref, dis_ref, dinv_ref):
    h0 = jnp.maximum(
        jnp.dot(x_ref[...], w0_ref[...], preferred_element_type=jnp.float32)
        + b0_ref[...], 0.0)
    t1 = jnp.dot(h0, wc0_ref[...], preferred_element_type=jnp.float32)
    deg = d0_ref[...] + d1_ref[...] + 1.0
    dis = 1.0 / jnp.sqrt(deg)
    dinv = 1.0 / deg
    t1s = t1 * dis
    ta_ref[...] = t1s[:, :HH]
    tb_ref[...] = t1s[:, HH:]
    tself_ref[...] = t1 * dinv
    dis_ref[...] = dis
    dinv_ref[...] = dinv


_k1_call = pl.pallas_call(
    _k1_body,
    grid=(N // ROWS_B,),
    in_specs=[
        pl.BlockSpec((ROWS_B, D_IN), lambda i: (i, 0)),
        pl.BlockSpec((D_IN, H), lambda i: (0, 0)),
        pl.BlockSpec((1, H), lambda i: (0, 0)),
        pl.BlockSpec((H, H), lambda i: (0, 0)),
        pl.BlockSpec((ROWS_B, 1), lambda i: (i, 0)),
        pl.BlockSpec((ROWS_B, 1), lambda i: (i, 0)),
    ],
    out_specs=[
        pl.BlockSpec((ROWS_B, HH), lambda i: (i, 0)),
        pl.BlockSpec((ROWS_B, HH), lambda i: (i, 0)),
        pl.BlockSpec((ROWS_B, H), lambda i: (i, 0)),
        pl.BlockSpec((ROWS_B, 1), lambda i: (i, 0)),
        pl.BlockSpec((ROWS_B, 1), lambda i: (i, 0)),
    ],
    out_shape=[
        jax.ShapeDtypeStruct((N, HH), jnp.float32),
        jax.ShapeDtypeStruct((N, HH), jnp.float32),
        jax.ShapeDtypeStruct((N, H), jnp.float32),
        jax.ShapeDtypeStruct((N, 1), jnp.float32),
        jax.ShapeDtypeStruct((N, 1), jnp.float32),
    ],
)


def _k4_body(p0_ref, p1_ref, tself_ref, dis_ref, dinv_ref, b_ref, w_ref,
             ta_ref, tb_ref, tself2_ref):
    p = jnp.concatenate([p0_ref[...], p1_ref[...]], axis=1)
    agg = p * dis_ref[...] + tself_ref[...] + b_ref[...]
    t2 = jnp.dot(agg, w_ref[...], preferred_element_type=jnp.float32)
    t2s = t2 * dis_ref[...]
    ta_ref[...] = t2s[:, :HH]
    tb_ref[...] = t2s[:, HH:]
    tself2_ref[...] = t2 * dinv_ref[...]


_k4_call = pl.pallas_call(
    _k4_body,
    grid=(N // ROWS_B,),
    in_specs=[
        pl.BlockSpec((ROWS_B, HH), lambda i: (i, 0)),
        pl.BlockSpec((ROWS_B, HH), lambda i: (i, 0)),
        pl.BlockSpec((ROWS_B, H), lambda i: (i, 0)),
        pl.BlockSpec((ROWS_B, 1), lambda i: (i, 0)),
        pl.BlockSpec((ROWS_B, 1), lambda i: (i, 0)),
        pl.BlockSpec((1, H), lambda i: (0, 0)),
        pl.BlockSpec((H, H), lambda i: (0, 0)),
    ],
    out_specs=[
        pl.BlockSpec((ROWS_B, HH), lambda i: (i, 0)),
        pl.BlockSpec((ROWS_B, HH), lambda i: (i, 0)),
        pl.BlockSpec((ROWS_B, H), lambda i: (i, 0)),
    ],
    out_shape=[
        jax.ShapeDtypeStruct((N, HH), jnp.float32),
        jax.ShapeDtypeStruct((N, HH), jnp.float32),
        jax.ShapeDtypeStruct((N, H), jnp.float32),
    ],
)


def _k6_body(q0_ref, q1_ref, tself_ref, dis_ref, b_ref, w_ref, bh_ref, out_ref):
    q = jnp.concatenate([q0_ref[...], q1_ref[...]], axis=1)
    agg = q * dis_ref[...] + tself_ref[...] + b_ref[...]
    out_ref[...] = (
        jnp.dot(agg, w_ref[...], preferred_element_type=jnp.float32) + bh_ref[...]
    )


_k6_call = pl.pallas_call(
    _k6_body,
    grid=(N // ROWS_B,),
    in_specs=[
        pl.BlockSpec((ROWS_B, HH), lambda i: (i, 0)),
        pl.BlockSpec((ROWS_B, HH), lambda i: (i, 0)),
        pl.BlockSpec((ROWS_B, H), lambda i: (i, 0)),
        pl.BlockSpec((ROWS_B, 1), lambda i: (i, 0)),
        pl.BlockSpec((1, H), lambda i: (0, 0)),
        pl.BlockSpec((H, N_OUT), lambda i: (0, 0)),
        pl.BlockSpec((1, N_OUT), lambda i: (0, 0)),
    ],
    out_specs=[pl.BlockSpec((ROWS_B, N_OUT), lambda i: (i, 0))],
    out_shape=[jax.ShapeDtypeStruct((N, N_OUT), jnp.float32)],
)


@jax.jit
def _run(edge_index, x, W0, b0, Wc0, bc0, Wc1, bc1, Wh, bh):
    pad = EP - E
    src_p = jnp.concatenate([edge_index[0], jnp.zeros((pad,), jnp.int32)])
    dst_p = jnp.concatenate([edge_index[1], jnp.full((pad,), N, jnp.int32)])
    src2 = src_p.reshape(NS, EPT)
    dst3 = dst_p.reshape(NS, NCHUNK, CHUNK)
    dstd = dst_p.reshape(NW, DNCHUNK, DCHUNK)
    zeros1 = jnp.zeros((NP,), jnp.float32)
    zeros2 = jnp.zeros((NP, HH), jnp.float32)

    degp = _deg_call(dstd, zeros1)
    d0 = degp[0, :N].reshape(N, 1)
    d1 = degp[1, :N].reshape(N, 1)

    t1a, t1b, tself1, dis, dinv = _k1_call(
        x, W0, b0.reshape(1, H), Wc0, d0, d1)

    p = _agg_call(src2, dst3, t1a, t1b, zeros2)
    t2a, t2b, tself2 = _k4_call(
        p[0, :N], p[1, :N], tself1, dis, dinv, bc0.reshape(1, H), Wc1)

    q = _agg_call(src2, dst3, t2a, t2b, zeros2)
    (out,) = _k6_call(
        q[0, :N], q[1, :N], tself2, dis, bc1.reshape(1, H), Wh,
        bh.reshape(1, N_OUT))
    return out


def kernel(edge_index, x, W0, b0, Wc0, bc0, Wc1, bc1, Wh, bh):
    return _run(edge_index, x, W0, b0, Wc0, bc0, Wc1, bc1, Wh, bh)


# speedup vs baseline: 2.1293x; 2.1293x over previous
import functools

import jax
import jax.numpy as jnp
from jax import lax
from jax.experimental import pallas as pl
from jax.experimental.pallas import tpu as pltpu
from jax.experimental.pallas import tpu_sc as plsc

N = 10000
E = 320000
D_IN = 128
H = 128
N_OUT = 2

NP = 10240
NC = 2
NS = 16
NW = NC * NS
EPW = E // NW
CHUNK = 80
NCHUNK = EPW // CHUNK
RPT = NP // NS

_sc_mesh = plsc.VectorSubcoreMesh(core_axis_name="c", subcore_axis_name="s")


def _deg_body(dst_hbm, out_hbm, didx_v, ones_v, zb_v, acc_sh):
    c = lax.axis_index("c")
    s = lax.axis_index("s")
    wid = s * NC + c
    pltpu.sync_copy(dst_hbm.at[wid], didx_v)
    for g in range(CHUNK // 16):
        ones_v[pl.ds(g * 16, 16)] = jnp.ones((16,), jnp.float32)
        zb_v[pl.ds(g * 16, 16)] = jnp.zeros((16,), jnp.float32)
    for k in range(RPT // CHUNK):
        pltpu.sync_copy(zb_v, acc_sh.at[pl.ds(s * RPT + k * CHUNK, CHUNK)])
    plsc.subcore_barrier()

    def step(i, carry):
        pltpu.sync_copy(ones_v, acc_sh.at[didx_v.at[i]], add=True)
        return carry

    lax.fori_loop(0, NCHUNK, step, 0)
    plsc.subcore_barrier()
    pltpu.sync_copy(acc_sh.at[pl.ds(s * RPT, RPT)], out_hbm.at[c, pl.ds(s * RPT, RPT)])


_deg_call = pl.kernel(
    _deg_body,
    out_type=jax.ShapeDtypeStruct((NC, NP), jnp.float32),
    mesh=_sc_mesh,
    scratch_types=[
        pltpu.VMEM((NCHUNK, CHUNK), jnp.int32),
        pltpu.VMEM((CHUNK,), jnp.float32),
        pltpu.VMEM((CHUNK,), jnp.float32),
        pltpu.VMEM_SHARED((NP,), jnp.float32),
    ],
)


def _agg_body(src_hbm, dst_hbm, table_hbm, out_hbm,
              sidx_v, didx_v, rows0_v, rows1_v, acc_sh, sem0, sem1):
    c = lax.axis_index("c")
    s = lax.axis_index("s")
    wid = s * NC + c
    pltpu.sync_copy(src_hbm.at[wid], sidx_v)
    pltpu.sync_copy(dst_hbm.at[wid], didx_v)

    def zrow(r, carry):
        for g in range(H // 16):
            rows0_v[r, pl.ds(g * 16, 16)] = jnp.zeros((16,), jnp.float32)
        return carry

    lax.fori_loop(0, CHUNK, zrow, 0)
    for k in range(RPT // CHUNK):
        pltpu.sync_copy(rows0_v, acc_sh.at[pl.ds(s * RPT + k * CHUNK, CHUNK)])
    plsc.subcore_barrier()

    def gather(i, buf, sem):
        return pltpu.async_copy(
            table_hbm.at[sidx_v.at[pl.ds(i * CHUNK, CHUNK)]], buf, sem)

    def wait(i, buf, sem):
        pltpu.make_async_copy(
            table_hbm.at[sidx_v.at[pl.ds(i * CHUNK, CHUNK)]], buf, sem).wait()

    def scat(i, buf):
        pltpu.sync_copy(buf, acc_sh.at[didx_v.at[i]], add=True)

    gather(0, rows0_v, sem0)

    def step(j, carry):
        i = 2 * j
        gather(i + 1, rows1_v, sem1)
        wait(i, rows0_v, sem0)
        scat(i, rows0_v)
        gather(i + 2, rows0_v, sem0)
        wait(i + 1, rows1_v, sem1)
        scat(i + 1, rows1_v)
        return carry

    lax.fori_loop(0, (NCHUNK - 1) // 2, step, 0)
    wait(NCHUNK - 1, rows0_v, sem0)
    scat(NCHUNK - 1, rows0_v)
    plsc.subcore_barrier()
    pltpu.sync_copy(acc_sh.at[pl.ds(s * RPT, RPT)], out_hbm.at[c, pl.ds(s * RPT, RPT)])


_agg_call = pl.kernel(
    _agg_body,
    out_type=jax.ShapeDtypeStruct((NC, NP, H), jnp.float32),
    mesh=_sc_mesh,
    scratch_types=[
        pltpu.VMEM((EPW,), jnp.int32),
        pltpu.VMEM((NCHUNK, CHUNK), jnp.int32),
        pltpu.VMEM((CHUNK, H), jnp.float32),
        pltpu.VMEM((CHUNK, H), jnp.float32),
        pltpu.VMEM_SHARED((NP, H), jnp.float32),
        pltpu.SemaphoreType.DMA,
        pltpu.SemaphoreType.DMA,
    ],
)

ROWS_B = 1000


def _k1_body(x_ref, w0_ref, b0_ref, wc0_ref, degp_ref,
             t1s_ref, tself_ref, dis_ref, dinv_ref):
    h0 = jnp.maximum(
        jnp.dot(x_ref[...], w0_ref[...], preferred_element_type=jnp.float32)
        + b0_ref[...], 0.0)
    t1 = jnp.dot(h0, wc0_ref[...], preferred_element_type=jnp.float32)
    deg = degp_ref[0] + degp_ref[1] + 1.0
    dis = 1.0 / jnp.sqrt(deg)
    dinv = 1.0 / deg
    t1s_ref[...] = t1 * dis
    tself_ref[...] = t1 * dinv
    dis_ref[...] = dis
    dinv_ref[...] = dinv


_k1_call = pl.pallas_call(
    _k1_body,
    grid=(N // ROWS_B,),
    in_specs=[
        pl.BlockSpec((ROWS_B, D_IN), lambda i: (i, 0)),
        pl.BlockSpec((D_IN, H), lambda i: (0, 0)),
        pl.BlockSpec((1, H), lambda i: (0, 0)),
        pl.BlockSpec((H, H), lambda i: (0, 0)),
        pl.BlockSpec((NC, ROWS_B, 1), lambda i: (0, i, 0)),
    ],
    out_specs=[
        pl.BlockSpec((ROWS_B, H), lambda i: (i, 0)),
        pl.BlockSpec((ROWS_B, H), lambda i: (i, 0)),
        pl.BlockSpec((ROWS_B, 1), lambda i: (i, 0)),
        pl.BlockSpec((ROWS_B, 1), lambda i: (i, 0)),
    ],
    out_shape=[
        jax.ShapeDtypeStruct((N, H), jnp.float32),
        jax.ShapeDtypeStruct((N, H), jnp.float32),
        jax.ShapeDtypeStruct((N, 1), jnp.float32),
        jax.ShapeDtypeStruct((N, 1), jnp.float32),
    ],
)


def _k4_body(p_ref, tself_ref, dis_ref, dinv_ref, b_ref, w_ref,
             t2s_ref, tself2_ref):
    agg = (p_ref[0] + p_ref[1]) * dis_ref[...] + tself_ref[...] + b_ref[...]
    t2 = jnp.dot(agg, w_ref[...], preferred_element_type=jnp.float32)
    t2s_ref[...] = t2 * dis_ref[...]
    tself2_ref[...] = t2 * dinv_ref[...]


_k4_call = pl.pallas_call(
    _k4_body,
    grid=(N // ROWS_B,),
    in_specs=[
        pl.BlockSpec((NC, ROWS_B, H), lambda i: (0, i, 0)),
        pl.BlockSpec((ROWS_B, H), lambda i: (i, 0)),
        pl.BlockSpec((ROWS_B, 1), lambda i: (i, 0)),
        pl.BlockSpec((ROWS_B, 1), lambda i: (i, 0)),
        pl.BlockSpec((1, H), lambda i: (0, 0)),
        pl.BlockSpec((H, H), lambda i: (0, 0)),
    ],
    out_specs=[
        pl.BlockSpec((ROWS_B, H), lambda i: (i, 0)),
        pl.BlockSpec((ROWS_B, H), lambda i: (i, 0)),
    ],
    out_shape=[
        jax.ShapeDtypeStruct((N, H), jnp.float32),
        jax.ShapeDtypeStruct((N, H), jnp.float32),
    ],
)


def _k6_body(q_ref, tself_ref, dis_ref, b_ref, w_ref, bh_ref, out_ref):
    agg = (q_ref[0] + q_ref[1]) * dis_ref[...] + tself_ref[...] + b_ref[...]
    out_ref[...] = (
        jnp.dot(agg, w_ref[...], preferred_element_type=jnp.float32) + bh_ref[...]
    )


_k6_call = pl.pallas_call(
    _k6_body,
    grid=(N // ROWS_B,),
    in_specs=[
        pl.BlockSpec((NC, ROWS_B, H), lambda i: (0, i, 0)),
        pl.BlockSpec((ROWS_B, H), lambda i: (i, 0)),
        pl.BlockSpec((ROWS_B, 1), lambda i: (i, 0)),
        pl.BlockSpec((1, H), lambda i: (0, 0)),
        pl.BlockSpec((H, N_OUT), lambda i: (0, 0)),
        pl.BlockSpec((1, N_OUT), lambda i: (0, 0)),
    ],
    out_specs=[pl.BlockSpec((ROWS_B, N_OUT), lambda i: (i, 0))],
    out_shape=[jax.ShapeDtypeStruct((N, N_OUT), jnp.float32)],
)


@jax.jit
def _run(edge_index, x, W0, b0, Wc0, bc0, Wc1, bc1, Wh, bh):
    src2 = edge_index[0].reshape(NW, EPW)
    dst3 = edge_index[1].reshape(NW, NCHUNK, CHUNK)

    degp = _deg_call(dst3).reshape(NC, NP, 1)

    t1s, tself1, dis, dinv = _k1_call(
        x, W0, b0.reshape(1, H), Wc0, degp)

    p = _agg_call(src2, dst3, t1s)
    t2s, tself2 = _k4_call(
        p, tself1, dis, dinv, bc0.reshape(1, H), Wc1)

    q = _agg_call(src2, dst3, t2s)
    (out,) = _k6_call(
        q, tself2, dis, bc1.reshape(1, H), Wh,
        bh.reshape(1, N_OUT))
    return out


def kernel(edge_index, x, W0, b0, Wc0, bc0, Wc1, bc1, Wh, bh):
    return _run(edge_index, x, W0, b0, Wc0, bc0, Wc1, bc1, Wh, bh)


# speedup vs baseline: 2.1904x; 1.0287x over previous
import functools

import jax
import jax.numpy as jnp
from jax import lax
from jax.experimental import pallas as pl
from jax.experimental.pallas import tpu as pltpu
from jax.experimental.pallas import tpu_sc as plsc

N = 10000
E = 320000
D_IN = 128
H = 128
N_OUT = 2

NP = 10240
NC = 2
NS = 16
NW = NC * NS
EPW = E // NW
CHUNK = 80
NCHUNK = EPW // CHUNK
RPT = NP // NS

_sc_mesh = plsc.VectorSubcoreMesh(core_axis_name="c", subcore_axis_name="s")


def _deg_body(e4_hbm, out_hbm, didx_v, ones_v, zb_v, acc_sh):
    c = lax.axis_index("c")
    s = lax.axis_index("s")
    wid = s * NC + c
    pltpu.sync_copy(e4_hbm.at[1, wid], didx_v)
    for g in range(CHUNK // 16):
        ones_v[pl.ds(g * 16, 16)] = jnp.ones((16,), jnp.float32)
        zb_v[pl.ds(g * 16, 16)] = jnp.zeros((16,), jnp.float32)
    for k in range(RPT // CHUNK):
        pltpu.sync_copy(zb_v, acc_sh.at[pl.ds(s * RPT + k * CHUNK, CHUNK)])
    plsc.subcore_barrier()

    def step(i, carry):
        pltpu.sync_copy(ones_v, acc_sh.at[didx_v.at[i]], add=True)
        return carry

    lax.fori_loop(0, NCHUNK, step, 0)
    plsc.subcore_barrier()
    pltpu.sync_copy(acc_sh.at[pl.ds(s * RPT, RPT)], out_hbm.at[c, pl.ds(s * RPT, RPT)])


_deg_call = pl.kernel(
    _deg_body,
    out_type=jax.ShapeDtypeStruct((NC, NP), jnp.float32),
    mesh=_sc_mesh,
    scratch_types=[
        pltpu.VMEM((NCHUNK, CHUNK), jnp.int32),
        pltpu.VMEM((CHUNK,), jnp.float32),
        pltpu.VMEM((CHUNK,), jnp.float32),
        pltpu.VMEM_SHARED((NP,), jnp.float32),
    ],
)


def _agg_body(e3_hbm, e4_hbm, table_hbm, out_hbm,
              sidx_v, didx_v, rows0_v, rows1_v, acc_sh, sem0, sem1):
    c = lax.axis_index("c")
    s = lax.axis_index("s")
    wid = s * NC + c
    pltpu.sync_copy(e3_hbm.at[0, wid], sidx_v)
    pltpu.sync_copy(e4_hbm.at[1, wid], didx_v)

    def zrow(r, carry):
        for g in range(H // 16):
            rows0_v[r, pl.ds(g * 16, 16)] = jnp.zeros((16,), jnp.float32)
        return carry

    lax.fori_loop(0, CHUNK, zrow, 0)
    for k in range(RPT // CHUNK):
        pltpu.sync_copy(rows0_v, acc_sh.at[pl.ds(s * RPT + k * CHUNK, CHUNK)])
    plsc.subcore_barrier()

    def gather(i, buf, sem):
        return pltpu.async_copy(
            table_hbm.at[sidx_v.at[pl.ds(i * CHUNK, CHUNK)]], buf, sem)

    def wait(i, buf, sem):
        pltpu.make_async_copy(
            table_hbm.at[sidx_v.at[pl.ds(i * CHUNK, CHUNK)]], buf, sem).wait()

    def scat(i, buf):
        pltpu.sync_copy(buf, acc_sh.at[didx_v.at[i]], add=True)

    gather(0, rows0_v, sem0)

    def step(j, carry):
        i = 2 * j
        gather(i + 1, rows1_v, sem1)
        wait(i, rows0_v, sem0)
        scat(i, rows0_v)
        gather(i + 2, rows0_v, sem0)
        wait(i + 1, rows1_v, sem1)
        scat(i + 1, rows1_v)
        return carry

    lax.fori_loop(0, (NCHUNK - 1) // 2, step, 0)
    wait(NCHUNK - 1, rows0_v, sem0)
    scat(NCHUNK - 1, rows0_v)
    plsc.subcore_barrier()
    pltpu.sync_copy(acc_sh.at[pl.ds(s * RPT, RPT)], out_hbm.at[c, pl.ds(s * RPT, RPT)])


_agg_call = pl.kernel(
    _agg_body,
    out_type=jax.ShapeDtypeStruct((NC, NP, H), jnp.float32),
    mesh=_sc_mesh,
    scratch_types=[
        pltpu.VMEM((EPW,), jnp.int32),
        pltpu.VMEM((NCHUNK, CHUNK), jnp.int32),
        pltpu.VMEM((CHUNK, H), jnp.float32),
        pltpu.VMEM((CHUNK, H), jnp.float32),
        pltpu.VMEM_SHARED((NP, H), jnp.float32),
        pltpu.SemaphoreType.DMA,
        pltpu.SemaphoreType.DMA,
    ],
)

ROWS_B = 1000


def _k1_body(x_ref, w0_ref, b0_ref, wc0_ref, degp_ref,
             t1s_ref, tself_ref):
    h0 = jnp.maximum(
        jnp.dot(x_ref[...], w0_ref[...], preferred_element_type=jnp.float32)
        + b0_ref[...], 0.0)
    t1 = jnp.dot(h0, wc0_ref[...], preferred_element_type=jnp.float32)
    deg = degp_ref[0] + degp_ref[1] + 1.0
    dis = 1.0 / jnp.sqrt(deg)
    dinv = 1.0 / deg
    t1s_ref[...] = t1 * dis
    tself_ref[...] = t1 * dinv


_k1_call = pl.pallas_call(
    _k1_body,
    grid=(N // ROWS_B,),
    in_specs=[
        pl.BlockSpec((ROWS_B, D_IN), lambda i: (i, 0)),
        pl.BlockSpec((D_IN, H), lambda i: (0, 0)),
        pl.BlockSpec((1, H), lambda i: (0, 0)),
        pl.BlockSpec((H, H), lambda i: (0, 0)),
        pl.BlockSpec((NC, ROWS_B, 1), lambda i: (0, i, 0)),
    ],
    out_specs=[
        pl.BlockSpec((ROWS_B, H), lambda i: (i, 0)),
        pl.BlockSpec((ROWS_B, H), lambda i: (i, 0)),
    ],
    out_shape=[
        jax.ShapeDtypeStruct((N, H), jnp.float32),
        jax.ShapeDtypeStruct((N, H), jnp.float32),
    ],
)


def _k4_body(p_ref, tself_ref, degp_ref, b_ref, w_ref,
             t2s_ref, tself2_ref):
    deg = degp_ref[0] + degp_ref[1] + 1.0
    dis = 1.0 / jnp.sqrt(deg)
    dinv = 1.0 / deg
    agg = (p_ref[0] + p_ref[1]) * dis + tself_ref[...] + b_ref[...]
    t2 = jnp.dot(agg, w_ref[...], preferred_element_type=jnp.float32)
    t2s_ref[...] = t2 * dis
    tself2_ref[...] = t2 * dinv


_k4_call = pl.pallas_call(
    _k4_body,
    grid=(N // ROWS_B,),
    in_specs=[
        pl.BlockSpec((NC, ROWS_B, H), lambda i: (0, i, 0)),
        pl.BlockSpec((ROWS_B, H), lambda i: (i, 0)),
        pl.BlockSpec((NC, ROWS_B, 1), lambda i: (0, i, 0)),
        pl.BlockSpec((1, H), lambda i: (0, 0)),
        pl.BlockSpec((H, H), lambda i: (0, 0)),
    ],
    out_specs=[
        pl.BlockSpec((ROWS_B, H), lambda i: (i, 0)),
        pl.BlockSpec((ROWS_B, H), lambda i: (i, 0)),
    ],
    out_shape=[
        jax.ShapeDtypeStruct((N, H), jnp.float32),
        jax.ShapeDtypeStruct((N, H), jnp.float32),
    ],
)


def _k6_body(q_ref, tself_ref, degp_ref, b_ref, w_ref, bh_ref, out_ref):
    deg = degp_ref[0] + degp_ref[1] + 1.0
    dis = 1.0 / jnp.sqrt(deg)
    agg = (q_ref[0] + q_ref[1]) * dis + tself_ref[...] + b_ref[...]
    out_ref[...] = (
        jnp.dot(agg, w_ref[...], preferred_element_type=jnp.float32) + bh_ref[...]
    )


_k6_call = pl.pallas_call(
    _k6_body,
    grid=(N // ROWS_B,),
    in_specs=[
        pl.BlockSpec((NC, ROWS_B, H), lambda i: (0, i, 0)),
        pl.BlockSpec((ROWS_B, H), lambda i: (i, 0)),
        pl.BlockSpec((NC, ROWS_B, 1), lambda i: (0, i, 0)),
        pl.BlockSpec((1, H), lambda i: (0, 0)),
        pl.BlockSpec((H, N_OUT), lambda i: (0, 0)),
        pl.BlockSpec((1, N_OUT), lambda i: (0, 0)),
    ],
    out_specs=[pl.BlockSpec((ROWS_B, N_OUT), lambda i: (i, 0))],
    out_shape=[jax.ShapeDtypeStruct((N, N_OUT), jnp.float32)],
)


@jax.jit
def _run(edge_index, x, W0, b0, Wc0, bc0, Wc1, bc1, Wh, bh):
    e3 = edge_index.reshape(2, NW, EPW)
    e4 = edge_index.reshape(2, NW, NCHUNK, CHUNK)

    degp = _deg_call(e4).reshape(NC, NP, 1)

    t1s, tself1 = _k1_call(x, W0, b0.reshape(1, H), Wc0, degp)

    p = _agg_call(e3, e4, t1s)
    t2s, tself2 = _k4_call(p, tself1, degp, bc0.reshape(1, H), Wc1)

    q = _agg_call(e3, e4, t2s)
    (out,) = _k6_call(
        q, tself2, degp, bc1.reshape(1, H), Wh, bh.reshape(1, N_OUT))
    return out


def kernel(edge_index, x, W0, b0, Wc0, bc0, Wc1, bc1, Wh, bh):
    return _run(edge_index, x, W0, b0, Wc0, bc0, Wc1, bc1, Wh, bh)
